# baseline (speedup 1.0000x reference)
import jax
import jax.numpy as jnp
from jax import lax
from jax.experimental import pallas as pl
from jax.experimental.pallas import tpu as pltpu
from jax.experimental.pallas import tpu_sc as plsc

VOCAB = 100000
EMBED = 128
B = 4096
L = 200

_NC = 2
_NS = 16
_NW = _NC * _NS
_N = B * L
_PER_W = _N // _NW
_CHUNK = 128
_NCHUNK = _PER_W // _CHUNK
_NBUF = 4
_MBUF = 2


def _emb_body(table_hbm, idx_hbm, out_hbm, idx_v, rows_v, rows_sh,
              sem_g, sem_x, sem_w):
    cid = lax.axis_index("c")
    sid = lax.axis_index("s")
    wid = sid * _NC + cid
    base = wid * _PER_W
    pltpu.sync_copy(idx_hbm.at[wid], idx_v)

    def gather(g, b):
        return pltpu.async_copy(table_hbm.at[idx_v.at[g]], rows_v.at[b],
                                sem_g.at[b])

    def xbar(b, m):
        return pltpu.async_copy(rows_v.at[b], rows_sh.at[sid].at[m],
                                sem_x.at[m])

    def dma(g, m):
        return pltpu.async_copy(rows_sh.at[sid].at[m],
                                out_hbm.at[pl.ds(base + g * _CHUNK, _CHUNK)],
                                sem_w.at[m])

    def wait_gather(b):
        pltpu.make_async_copy(table_hbm.at[idx_v.at[0]], rows_v.at[b],
                              sem_g.at[b]).wait()

    def wait_xbar(m):
        pltpu.make_async_copy(rows_v.at[0], rows_sh.at[sid].at[m],
                              sem_x.at[m]).wait()

    def wait_dma(m):
        pltpu.make_async_copy(rows_sh.at[sid].at[0],
                              out_hbm.at[pl.ds(base, _CHUNK)],
                              sem_w.at[m]).wait()

    gather(0, 0)
    gather(1, 1)

    def step(t, carry):
        g0 = _NBUF * t
        for j in range(_NBUF):
            g = g0 + j
            wait_gather(j)
            m = j % _MBUF

            @pl.when(g >= _MBUF)
            def _(m=m):
                wait_dma(m)

            xbar(j, m)
            m1 = (m + _MBUF - 1) % _MBUF

            @pl.when(g >= 1)
            def _(g=g, m1=m1):
                wait_xbar(m1)
                dma(g - 1, m1)

            b2 = (j + 2) % _NBUF

            @pl.when(g + 2 < _NCHUNK)
            def _(g=g, b2=b2):
                gather(g + 2, b2)

        return carry

    lax.fori_loop(0, _NCHUNK // _NBUF, step, 0, unroll=False)
    wait_xbar((_NCHUNK - 1) % _MBUF)
    dma(_NCHUNK - 1, (_NCHUNK - 1) % _MBUF)
    for m in range(_MBUF):
        wait_dma(m)


@jax.jit
def kernel(x, table):
    idx = x.reshape(_NW, _NCHUNK, _CHUNK).astype(jnp.int32)
    mesh = plsc.VectorSubcoreMesh(core_axis_name="c", subcore_axis_name="s")
    out = pl.kernel(
        _emb_body,
        out_type=jax.ShapeDtypeStruct((_N, EMBED), jnp.float32),
        mesh=mesh,
        scratch_types=[
            pltpu.VMEM((_NCHUNK, _CHUNK), jnp.int32),
            pltpu.VMEM((_NBUF, _CHUNK, EMBED), jnp.float32),
            pltpu.VMEM_SHARED((_NS, _MBUF, _CHUNK, EMBED), jnp.float32),
            pltpu.SemaphoreType.DMA((_NBUF,)),
            pltpu.SemaphoreType.DMA((_MBUF,)),
            pltpu.SemaphoreType.DMA((_MBUF,)),
        ],
    )(table, idx)
    return out.reshape(B, L, EMBED)
